# baseline (speedup 1.0000x reference)
import numpy as np
import jax
import jax.numpy as jnp
from jax.experimental import pallas as pl
from jax.experimental.pallas import tpu as pltpu

_B, _C_IN, _C_OUT, _H, _W = 4, 32, 32, 64, 64
_K, _AREA, _MLP = 32, 9, 16
_KM_ITERS = 5
_N = _H * _W
_PADL = 128

_INIT_IDX = np.linspace(0, _N - 1, _K).astype(np.int32)
_TAPS = [dh * _W + dw for dh in (-1, 0, 1) for dw in (-1, 0, 1)]


def _canconv_kernel(xT_ref, kh_w1_ref, kh_b1_ref, kh_w2_ref, kh_b2_ref,
                    area_w_ref, area_b_ref, cin_w_ref, cin_b_ref, cout_w_ref,
                    cout_b_ref, krT_ref, bias_w1_ref, bias_b1_ref, bias_w2_ref,
                    bias_b2_ref, bias_w3_ref, bias_b3_ref, out_ref, fpad_ref):
    fpad_ref[:, :_PADL] = jnp.zeros((_B * _C_IN, _PADL), jnp.float32)
    fpad_ref[:, pl.ds(_PADL + _N, _PADL)] = jnp.zeros(
        (_B * _C_IN, _PADL), jnp.float32)
    for bb in range(_B):
        fpad_ref[bb * _C_IN:(bb + 1) * _C_IN, pl.ds(_PADL, _N)] = xT_ref[bb]

    iota_s = jax.lax.broadcasted_iota(jnp.int32, (_K, _N), 0)
    ones_n1 = jnp.ones((_N, 1), jnp.float32)
    dsel = (jax.lax.broadcasted_iota(jnp.int32, (_K, _N), 0) * (_N - 1)
            - jax.lax.broadcasted_iota(jnp.int32, (_K, _N), 1) * (_K - 1))
    selT = ((dsel >= 0) & (dsel <= _K - 2)).astype(jnp.float32)
    col = jax.lax.broadcasted_iota(jnp.int32, (1, _N), 1) % _W
    mask_l = (col != 0).astype(jnp.float32)
    mask_r = (col != _W - 1).astype(jnp.float32)

    krT = krT_ref[:]

    for b in range(_B):
        featT = xT_ref[b]
        f2 = jnp.sum(featT * featT, axis=0, keepdims=True)

        centroids = jax.lax.dot_general(
            selT, featT, (((1,), (1,)), ((), ())),
            preferred_element_type=jnp.float32,
            precision=jax.lax.Precision.HIGHEST)

        ohT = None
        for _ in range(_KM_ITERS):
            sT = jnp.dot(centroids, featT,
                         preferred_element_type=jnp.float32)
            c2 = jnp.sum(centroids * centroids, axis=1,
                         keepdims=True)
            dT = (f2 - 2.0 * sT) + c2
            dmin = jnp.min(dT, axis=0, keepdims=True)
            idxr = jnp.min(jnp.where(dT == dmin, iota_s, _K), axis=0,
                           keepdims=True)
            ohT = (iota_s == idxr).astype(jnp.float32)
            counts = jnp.dot(ohT, ones_n1,
                             preferred_element_type=jnp.float32)
            sums = jax.lax.dot_general(
                ohT, featT, (((1,), (1,)), ((), ())),
                preferred_element_type=jnp.float32)
            centroids = sums / jnp.maximum(counts, 1.0)

        kf = jax.nn.relu(
            jnp.dot(centroids, kh_w1_ref[:],
                    preferred_element_type=jnp.float32) + kh_b1_ref[:])
        kf = jax.nn.relu(
            jnp.dot(kf, kh_w2_ref[:],
                    preferred_element_type=jnp.float32) + kh_b2_ref[:])
        w_cin = jax.nn.sigmoid(
            jnp.dot(kf, cin_w_ref[:],
                    preferred_element_type=jnp.float32) + cin_b_ref[:])
        w_area = jax.nn.sigmoid(
            jnp.dot(kf, area_w_ref[:],
                    preferred_element_type=jnp.float32) + area_b_ref[:])
        w_cout = jax.nn.sigmoid(
            jnp.dot(kf, cout_w_ref[:],
                    preferred_element_type=jnp.float32) + cout_b_ref[:])
        bf = jax.nn.relu(
            jnp.dot(centroids, bias_w1_ref[:],
                    preferred_element_type=jnp.float32) + bias_b1_ref[:])
        bf = jax.nn.relu(
            jnp.dot(bf, bias_w2_ref[:],
                    preferred_element_type=jnp.float32) + bias_b2_ref[:])
        bias_c = (jnp.dot(bf, bias_w3_ref[:],
                          preferred_element_type=jnp.float32)
                  + bias_b3_ref[:])

        def gatherT(w):
            return jax.lax.dot_general(
                w, ohT, (((0,), (0,)), ((), ())),
                preferred_element_type=jnp.float32)

        cin_pxT = gatherT(w_cin)
        area_pxT = gatherT(w_area)
        cout_pxT = gatherT(w_cout)
        bias_pxT = gatherT(bias_c)

        parts = []
        for a, off in enumerate(_TAPS):
            xs = (featT if off == 0
                  else fpad_ref[b * _C_IN:(b + 1) * _C_IN,
                                pl.ds(_PADL + off, _N)])
            scale = area_pxT[a:a + 1, :]
            dw = (a % 3) - 1
            if dw == -1:
                scale = scale * mask_l
            elif dw == 1:
                scale = scale * mask_r
            parts.append(xs * cin_pxT * scale)
        patchesT = jnp.concatenate(parts, axis=0)
        preT = jnp.dot(krT, patchesT,
                       preferred_element_type=jnp.float32)
        out_ref[b] = preT * cout_pxT + bias_pxT


def kernel(x, kh_w1, kh_b1, kh_w2, kh_b2, area_w, area_b, cin_w, cin_b,
           cout_w, cout_b, kernels, bias_w1, bias_b1, bias_w2, bias_b2,
           bias_w3, bias_b3):
    b, c, h, w = x.shape
    n = h * w
    xT = x.reshape(b, c, n)
    krT = kernels.transpose(2, 1, 0).reshape(_C_OUT, _AREA * _C_IN)

    row = lambda v: v.reshape(1, -1)
    full = lambda shape: pl.BlockSpec(shape, lambda: tuple(0 for _ in shape))

    out = pl.pallas_call(
        _canconv_kernel,
        in_specs=[
            full((b, c, n)),
            full((c, _MLP)), full((1, _MLP)),
            full((_MLP, _MLP)), full((1, _MLP)),
            full((_MLP, _AREA)), full((1, _AREA)),
            full((_MLP, c)), full((1, c)),
            full((_MLP, _C_OUT)), full((1, _C_OUT)),
            full((_C_OUT, _AREA * _C_IN)),
            full((c, _MLP)), full((1, _MLP)),
            full((_MLP, _MLP)), full((1, _MLP)),
            full((_MLP, _C_OUT)), full((1, _C_OUT)),
        ],
        out_specs=full((b, _C_OUT, n)),
        out_shape=jax.ShapeDtypeStruct((b, _C_OUT, n), jnp.float32),
        scratch_shapes=[pltpu.VMEM((b * c, n + 2 * _PADL), jnp.float32)],
    )(xT, kh_w1, row(kh_b1), kh_w2, row(kh_b2), area_w, row(area_b),
      cin_w, row(cin_b), cout_w, row(cout_b), krT, bias_w1, row(bias_b1),
      bias_w2, row(bias_b2), bias_w3, row(bias_b3))
    return out.reshape(b, _C_OUT, h, w)


# speedup vs baseline: 1.0187x; 1.0187x over previous
import numpy as np
import jax
import jax.numpy as jnp
from jax.experimental import pallas as pl
from jax.experimental.pallas import tpu as pltpu

_B, _C_IN, _C_OUT, _H, _W = 4, 32, 32, 64, 64
_K, _AREA, _MLP = 32, 9, 16
_KM_ITERS = 5
_N = _H * _W
_PADL = 128

_INIT_IDX = np.linspace(0, _N - 1, _K).astype(np.int32)
_TAPS = [dh * _W + dw for dh in (-1, 0, 1) for dw in (-1, 0, 1)]


def _canconv_kernel(xT_ref, kh_w1_ref, kh_b1_ref, kh_w2_ref, kh_b2_ref,
                    area_w_ref, area_b_ref, cin_w_ref, cin_b_ref, cout_w_ref,
                    cout_b_ref, krT_ref, bias_w1_ref, bias_b1_ref, bias_w2_ref,
                    bias_b2_ref, bias_w3_ref, bias_b3_ref, out_ref, fpad_ref):
    fpad_ref[:, :_PADL] = jnp.zeros((_B * _C_IN, _PADL), jnp.float32)
    fpad_ref[:, pl.ds(_PADL + _N, _PADL)] = jnp.zeros(
        (_B * _C_IN, _PADL), jnp.float32)
    for bb in range(_B):
        fpad_ref[bb * _C_IN:(bb + 1) * _C_IN, pl.ds(_PADL, _N)] = xT_ref[bb]

    iota_s = jax.lax.broadcasted_iota(jnp.int32, (_K, _N), 0)
    dsel = (jax.lax.broadcasted_iota(jnp.int32, (_K, _N), 0) * (_N - 1)
            - jax.lax.broadcasted_iota(jnp.int32, (_K, _N), 1) * (_K - 1))
    selT = ((dsel >= 0) & (dsel <= _K - 2)).astype(jnp.float32)
    col = jax.lax.broadcasted_iota(jnp.int32, (1, _N), 1) % _W
    mask_l = (col != 0).astype(jnp.float32)
    mask_r = (col != _W - 1).astype(jnp.float32)

    krT = krT_ref[:]

    ones_1n = jnp.ones((1, _N), jnp.float32)
    for b in range(_B):
        featT = xT_ref[b]
        featT_ext = jnp.concatenate([featT, ones_1n], axis=0)
        f2 = jnp.sum(featT * featT, axis=0, keepdims=True)

        centroids = jax.lax.dot_general(
            selT, featT, (((1,), (1,)), ((), ())),
            preferred_element_type=jnp.float32,
            precision=jax.lax.Precision.HIGHEST)

        ohT = None
        for _ in range(_KM_ITERS):
            sT = jnp.dot(centroids, featT,
                         preferred_element_type=jnp.float32)
            c2 = jnp.sum(centroids * centroids, axis=1,
                         keepdims=True)
            dT = (f2 - 2.0 * sT) + c2
            dmin = jnp.min(dT, axis=0, keepdims=True)
            idxr = jnp.min(jnp.where(dT == dmin, iota_s, _K), axis=0,
                           keepdims=True)
            ohT = (iota_s == idxr).astype(jnp.float32)
            sums_ext = jax.lax.dot_general(
                ohT, featT_ext, (((1,), (1,)), ((), ())),
                preferred_element_type=jnp.float32)
            counts = sums_ext[:, _C_IN:_C_IN + 1]
            centroids = sums_ext[:, :_C_IN] / jnp.maximum(counts, 1.0)

        kf = jax.nn.relu(
            jnp.dot(centroids, kh_w1_ref[:],
                    preferred_element_type=jnp.float32) + kh_b1_ref[:])
        kf = jax.nn.relu(
            jnp.dot(kf, kh_w2_ref[:],
                    preferred_element_type=jnp.float32) + kh_b2_ref[:])
        w_cin = jax.nn.sigmoid(
            jnp.dot(kf, cin_w_ref[:],
                    preferred_element_type=jnp.float32) + cin_b_ref[:])
        w_area = jax.nn.sigmoid(
            jnp.dot(kf, area_w_ref[:],
                    preferred_element_type=jnp.float32) + area_b_ref[:])
        w_cout = jax.nn.sigmoid(
            jnp.dot(kf, cout_w_ref[:],
                    preferred_element_type=jnp.float32) + cout_b_ref[:])
        bf = jax.nn.relu(
            jnp.dot(centroids, bias_w1_ref[:],
                    preferred_element_type=jnp.float32) + bias_b1_ref[:])
        bf = jax.nn.relu(
            jnp.dot(bf, bias_w2_ref[:],
                    preferred_element_type=jnp.float32) + bias_b2_ref[:])
        bias_c = (jnp.dot(bf, bias_w3_ref[:],
                          preferred_element_type=jnp.float32)
                  + bias_b3_ref[:])

        w_all = jnp.concatenate(
            [w_cin, w_area, jnp.zeros((_K, 7), jnp.float32), w_cout,
             bias_c], axis=1)
        px_all = jax.lax.dot_general(
            w_all, ohT, (((0,), (0,)), ((), ())),
            preferred_element_type=jnp.float32)
        cin_pxT = px_all[0:_C_IN, :]
        area_pxT = px_all[_C_IN:_C_IN + _AREA, :]
        cout_pxT = px_all[48:48 + _C_OUT, :]
        bias_pxT = px_all[80:80 + _C_OUT, :]

        parts = []
        for a, off in enumerate(_TAPS):
            xs = (featT if off == 0
                  else fpad_ref[b * _C_IN:(b + 1) * _C_IN,
                                pl.ds(_PADL + off, _N)])
            scale = area_pxT[a:a + 1, :]
            dw = (a % 3) - 1
            if dw == -1:
                scale = scale * mask_l
            elif dw == 1:
                scale = scale * mask_r
            parts.append(xs * cin_pxT * scale)
        patchesT = jnp.concatenate(parts, axis=0)
        preT = jnp.dot(krT, patchesT,
                       preferred_element_type=jnp.float32)
        out_ref[b] = preT * cout_pxT + bias_pxT


def kernel(x, kh_w1, kh_b1, kh_w2, kh_b2, area_w, area_b, cin_w, cin_b,
           cout_w, cout_b, kernels, bias_w1, bias_b1, bias_w2, bias_b2,
           bias_w3, bias_b3):
    b, c, h, w = x.shape
    n = h * w
    xT = x.reshape(b, c, n)
    krT = kernels.transpose(2, 1, 0).reshape(_C_OUT, _AREA * _C_IN)

    row = lambda v: v.reshape(1, -1)
    full = lambda shape: pl.BlockSpec(shape, lambda: tuple(0 for _ in shape))

    out = pl.pallas_call(
        _canconv_kernel,
        in_specs=[
            full((b, c, n)),
            full((c, _MLP)), full((1, _MLP)),
            full((_MLP, _MLP)), full((1, _MLP)),
            full((_MLP, _AREA)), full((1, _AREA)),
            full((_MLP, c)), full((1, c)),
            full((_MLP, _C_OUT)), full((1, _C_OUT)),
            full((_C_OUT, _AREA * _C_IN)),
            full((c, _MLP)), full((1, _MLP)),
            full((_MLP, _MLP)), full((1, _MLP)),
            full((_MLP, _C_OUT)), full((1, _C_OUT)),
        ],
        out_specs=full((b, _C_OUT, n)),
        out_shape=jax.ShapeDtypeStruct((b, _C_OUT, n), jnp.float32),
        scratch_shapes=[pltpu.VMEM((b * c, n + 2 * _PADL), jnp.float32)],
    )(xT, kh_w1, row(kh_b1), kh_w2, row(kh_b2), area_w, row(area_b),
      cin_w, row(cin_b), cout_w, row(cout_b), krT, bias_w1, row(bias_b1),
      bias_w2, row(bias_b2), bias_w3, row(bias_b3))
    return out.reshape(b, _C_OUT, h, w)


# speedup vs baseline: 1.0511x; 1.0319x over previous
import numpy as np
import jax
import jax.numpy as jnp
from jax.experimental import pallas as pl
from jax.experimental.pallas import tpu as pltpu

_B, _C_IN, _C_OUT, _H, _W = 4, 32, 32, 64, 64
_K, _AREA, _MLP = 32, 9, 16
_KM_ITERS = 5
_N = _H * _W
_PADL = 128

_INIT_IDX = np.linspace(0, _N - 1, _K).astype(np.int32)
_TAPS = [dh * _W + dw for dh in (-1, 0, 1) for dw in (-1, 0, 1)]


def _canconv_kernel(xT_ref, kh_w1_ref, kh_b1_ref, kh_w2_ref, kh_b2_ref,
                    area_w_ref, area_b_ref, cin_w_ref, cin_b_ref, cout_w_ref,
                    cout_b_ref, krT_ref, bias_w1_ref, bias_b1_ref, bias_w2_ref,
                    bias_b2_ref, bias_w3_ref, bias_b3_ref, out_ref, fpad_ref):
    fpad_ref[:, :_PADL] = jnp.zeros((_B * _C_IN, _PADL), jnp.float32)
    fpad_ref[:, pl.ds(_PADL + _N, _PADL)] = jnp.zeros(
        (_B * _C_IN, _PADL), jnp.float32)
    for bb in range(_B):
        fpad_ref[bb * _C_IN:(bb + 1) * _C_IN, pl.ds(_PADL, _N)] = xT_ref[bb]

    iota_s = jax.lax.broadcasted_iota(jnp.int32, (_K, _N), 0)
    dsel = (jax.lax.broadcasted_iota(jnp.int32, (_K, _N), 0) * (_N - 1)
            - jax.lax.broadcasted_iota(jnp.int32, (_K, _N), 1) * (_K - 1))
    selT = ((dsel >= 0) & (dsel <= _K - 2)).astype(jnp.float32)
    col = jax.lax.broadcasted_iota(jnp.int32, (1, _N), 1) % _W
    mask_l = (col != 0).astype(jnp.float32)
    mask_r = (col != _W - 1).astype(jnp.float32)

    krT = krT_ref[:]

    ones_1n = jnp.ones((1, _N), jnp.float32)
    for b in range(_B):
        featT = xT_ref[b]
        featT_ext = jnp.concatenate([featT, ones_1n], axis=0)
        f2 = jnp.sum(featT * featT, axis=0, keepdims=True)

        centroids = jax.lax.dot_general(
            selT, featT, (((1,), (1,)), ((), ())),
            preferred_element_type=jnp.float32,
            precision=jax.lax.Precision.HIGHEST)

        ohT = None
        for _ in range(_KM_ITERS):
            sT = jnp.dot(centroids, featT,
                         preferred_element_type=jnp.float32)
            c2 = jnp.sum(centroids * centroids, axis=1,
                         keepdims=True)
            dT = (f2 - 2.0 * sT) + c2
            dmin = jnp.min(dT, axis=0, keepdims=True)
            idxr = jnp.min(jnp.where(dT == dmin, iota_s, _K), axis=0,
                           keepdims=True)
            ohT = (iota_s == idxr).astype(jnp.float32)
            sums_ext = jax.lax.dot_general(
                ohT, featT_ext, (((1,), (1,)), ((), ())),
                preferred_element_type=jnp.float32)
            counts = sums_ext[:, _C_IN:_C_IN + 1]
            centroids = sums_ext[:, :_C_IN] / jnp.maximum(counts, 1.0)

        kf = jax.nn.relu(
            jnp.dot(centroids, kh_w1_ref[:],
                    preferred_element_type=jnp.float32) + kh_b1_ref[:])
        kf = jax.nn.relu(
            jnp.dot(kf, kh_w2_ref[:],
                    preferred_element_type=jnp.float32) + kh_b2_ref[:])
        w_cin = jax.nn.sigmoid(
            jnp.dot(kf, cin_w_ref[:],
                    preferred_element_type=jnp.float32) + cin_b_ref[:])
        w_area = jax.nn.sigmoid(
            jnp.dot(kf, area_w_ref[:],
                    preferred_element_type=jnp.float32) + area_b_ref[:])
        w_cout = jax.nn.sigmoid(
            jnp.dot(kf, cout_w_ref[:],
                    preferred_element_type=jnp.float32) + cout_b_ref[:])
        bf = jax.nn.relu(
            jnp.dot(centroids, bias_w1_ref[:],
                    preferred_element_type=jnp.float32) + bias_b1_ref[:])
        bf = jax.nn.relu(
            jnp.dot(bf, bias_w2_ref[:],
                    preferred_element_type=jnp.float32) + bias_b2_ref[:])
        bias_c = (jnp.dot(bf, bias_w3_ref[:],
                          preferred_element_type=jnp.float32)
                  + bias_b3_ref[:])

        w_all = jnp.concatenate(
            [w_cin, w_area, jnp.zeros((_K, 7), jnp.float32), w_cout,
             bias_c], axis=1)
        px_all = jax.lax.dot_general(
            w_all, ohT, (((0,), (0,)), ((), ())),
            preferred_element_type=jnp.float32)
        cin_pxT = px_all[0:_C_IN, :]
        area_pxT = px_all[_C_IN:_C_IN + _AREA, :]
        cout_pxT = px_all[48:48 + _C_OUT, :]
        bias_pxT = px_all[80:80 + _C_OUT, :]

        parts = []
        for a, off in enumerate(_TAPS):
            xs = (featT if off == 0
                  else fpad_ref[b * _C_IN:(b + 1) * _C_IN,
                                pl.ds(_PADL + off, _N)])
            scale = area_pxT[a:a + 1, :]
            dw = (a % 3) - 1
            if dw == -1:
                scale = scale * mask_l
            elif dw == 1:
                scale = scale * mask_r
            parts.append((xs * cin_pxT * scale).astype(jnp.bfloat16))
        patchesT = jnp.concatenate(parts, axis=0)
        preT = jnp.dot(krT.astype(jnp.bfloat16), patchesT,
                       preferred_element_type=jnp.float32)
        out_ref[b] = preT * cout_pxT + bias_pxT


def kernel(x, kh_w1, kh_b1, kh_w2, kh_b2, area_w, area_b, cin_w, cin_b,
           cout_w, cout_b, kernels, bias_w1, bias_b1, bias_w2, bias_b2,
           bias_w3, bias_b3):
    b, c, h, w = x.shape
    n = h * w
    xT = x.reshape(b, c, n)
    krT = kernels.transpose(2, 1, 0).reshape(_C_OUT, _AREA * _C_IN)

    row = lambda v: v.reshape(1, -1)
    full = lambda shape: pl.BlockSpec(shape, lambda: tuple(0 for _ in shape))

    out = pl.pallas_call(
        _canconv_kernel,
        in_specs=[
            full((b, c, n)),
            full((c, _MLP)), full((1, _MLP)),
            full((_MLP, _MLP)), full((1, _MLP)),
            full((_MLP, _AREA)), full((1, _AREA)),
            full((_MLP, c)), full((1, c)),
            full((_MLP, _C_OUT)), full((1, _C_OUT)),
            full((_C_OUT, _AREA * _C_IN)),
            full((c, _MLP)), full((1, _MLP)),
            full((_MLP, _MLP)), full((1, _MLP)),
            full((_MLP, _C_OUT)), full((1, _C_OUT)),
        ],
        out_specs=full((b, _C_OUT, n)),
        out_shape=jax.ShapeDtypeStruct((b, _C_OUT, n), jnp.float32),
        scratch_shapes=[pltpu.VMEM((b * c, n + 2 * _PADL), jnp.float32)],
    )(xT, kh_w1, row(kh_b1), kh_w2, row(kh_b2), area_w, row(area_b),
      cin_w, row(cin_b), cout_w, row(cout_b), krT, bias_w1, row(bias_b1),
      bias_w2, row(bias_b2), bias_w3, row(bias_b3))
    return out.reshape(b, _C_OUT, h, w)
